# Initial kernel scaffold; baseline (speedup 1.0000x reference)
import jax
import jax.numpy as jnp
from jax.experimental import pallas as pl


def kernel(x, edge_index, W1_l, b1_l, W1_r, W2_l, b2_l, W2_r, W_lin, b_lin):
    raise NotImplementedError("write your pallas kernel here")


# speedup vs baseline: 6.6381x; 6.6381x over previous
import functools

import jax
import jax.numpy as jnp
from jax import lax
from jax.experimental import pallas as pl
from jax.experimental.pallas import tpu as pltpu
from jax.experimental.pallas import tpu_sc as plsc

N = 10000
E = 320000
D = 128

K = 128
NC = 2
NS = 16
NW = NC * NS
CHUNKS = E // K
ITERS = (CHUNKS + NW - 1) // NW
ZR = 40

def _make_segsum(with_deg):
    mesh = plsc.VectorSubcoreMesh(core_axis_name="c", subcore_axis_name="s")
    out_type = [jax.ShapeDtypeStruct((NC * N, D), jnp.float32)]
    if with_deg:
        out_type.append(jax.ShapeDtypeStruct((NC * N,), jnp.float32))
    scratch = [
        pltpu.VMEM((K,), jnp.int32),
        pltpu.VMEM((K,), jnp.int32),
        pltpu.VMEM((K, D), jnp.float32),
        pltpu.VMEM((K,), jnp.float32),
        pltpu.VMEM((ZR, D), jnp.float32),
        pltpu.VMEM((640,), jnp.float32),
        pltpu.VMEM_SHARED((N, D), jnp.float32),
        pltpu.VMEM_SHARED((N,), jnp.float32),
        pltpu.SemaphoreType.DMA,
    ]

    def body(x_hbm, src_hbm, dst_hbm, *refs):
        if with_deg:
            out, deg_out = refs[0], refs[1]
            rest = refs[2:]
        else:
            out = refs[0]
            rest = refs[1:]
        srcv, dstv, rows, ones, zbuf, zdeg, acc, dacc, sem = rest

        cid = lax.axis_index("c")
        sid = lax.axis_index("s")
        w = sid * NC + cid

        _zeros16 = jnp.zeros((16,), jnp.float32)
        _ones16 = jnp.ones((16,), jnp.float32)

        for r in range(ZR):
            for c in range(8):
                zbuf[r, pl.ds(c * 16, 16)] = _zeros16
        for i in range(640 // 16):
            zdeg[pl.ds(i * 16, 16)] = _zeros16
        if with_deg:
            for i in range(K // 16):
                ones[pl.ds(i * 16, 16)] = _ones16

        @pl.when(sid < 15)
        def _():
            def zrow(i, carry):
                pltpu.sync_copy(zbuf, acc.at[pl.ds(sid * 640 + i * ZR, ZR)])
                return carry
            lax.fori_loop(0, 16, zrow, 0)
            if with_deg:
                pltpu.sync_copy(zdeg, dacc.at[pl.ds(sid * 640, 640)])

        @pl.when(sid == 15)
        def _():
            def zrow(i, carry):
                pltpu.sync_copy(zbuf, acc.at[pl.ds(9600 + i * ZR, ZR)])
                return carry
            lax.fori_loop(0, 10, zrow, 0)
            if with_deg:
                pltpu.sync_copy(zdeg.at[pl.ds(0, 400)], dacc.at[pl.ds(9600, 400)])

        plsc.subcore_barrier()

        def step(it, carry):
            chunk = w + it * NW

            @pl.when(chunk < CHUNKS)
            def _():
                base = chunk * K
                pltpu.sync_copy(src_hbm.at[pl.ds(base, K)], srcv)
                pltpu.sync_copy(dst_hbm.at[pl.ds(base, K)], dstv)
                pltpu.async_copy(x_hbm.at[srcv], rows, sem).wait()
                pltpu.sync_copy(rows, acc.at[dstv], add=True)
                if with_deg:
                    pltpu.sync_copy(ones, dacc.at[dstv], add=True)

            return carry

        lax.fori_loop(0, ITERS, step, 0)

        plsc.subcore_barrier()

        @pl.when(sid < 15)
        def _():
            pltpu.sync_copy(acc.at[pl.ds(sid * 640, 640)],
                            out.at[pl.ds(cid * N + sid * 640, 640)])
            if with_deg:
                pltpu.sync_copy(dacc.at[pl.ds(sid * 640, 640)], zdeg)
                pltpu.sync_copy(zdeg,
                                deg_out.at[pl.ds(cid * N + sid * 640, 640)])

        @pl.when(sid == 15)
        def _():
            pltpu.sync_copy(acc.at[pl.ds(9600, 400)],
                            out.at[pl.ds(cid * N + 9600, 400)])
            if with_deg:
                pltpu.sync_copy(dacc.at[pl.ds(9600, 400)],
                                zdeg.at[pl.ds(0, 400)])
                pltpu.sync_copy(zdeg.at[pl.ds(0, 400)],
                                deg_out.at[pl.ds(cid * N + 9600, 400)])

    return pl.kernel(body, mesh=mesh, out_type=out_type, scratch_types=scratch)


_segsum_deg = _make_segsum(True)
_segsum = _make_segsum(False)


_CD = (((1,), (1,)), ((), ()))


def _tc1_body(a0, a1, d0, d1, x, wl, wr, b, o):
    deg = jnp.maximum(d0[...] + d1[...], 1.0)
    mean = (a0[...] + a1[...]) / deg
    y = lax.dot_general(mean, wl[...], _CD, preferred_element_type=jnp.float32)
    y = y + lax.dot_general(x[...], wr[...], _CD, preferred_element_type=jnp.float32)
    o[...] = jnp.maximum(y + b[...], 0.0)


def _tc2_body(m0, m1, d0, d1, x1, wl, wr, b, wa, wb, bl, o):
    deg = jnp.maximum(d0[...] + d1[...], 1.0)
    mean = (m0[...] + m1[...]) / deg
    y = lax.dot_general(mean, wl[...], _CD, preferred_element_type=jnp.float32)
    y = y + lax.dot_general(x1[...], wr[...], _CD, preferred_element_type=jnp.float32)
    x2 = jnp.maximum(y + b[...], 0.0)
    z = lax.dot_general(x1[...], wa[...], _CD, preferred_element_type=jnp.float32)
    z = z + lax.dot_general(x2, wb[...], _CD, preferred_element_type=jnp.float32)
    o[...] = z + bl[...]


BN = 1000


def _row_spec():
    return pl.BlockSpec((BN, D), lambda i: (i, 0))


def _deg_spec():
    return pl.BlockSpec((BN, 1), lambda i: (i, 0))


def _w_spec():
    return pl.BlockSpec((D, D), lambda i: (0, 0))


def _b_spec():
    return pl.BlockSpec((1, D), lambda i: (0, 0))


def _tc1(a0, a1, d0, d1, x, wl, wr, b):
    return pl.pallas_call(
        _tc1_body,
        grid=(N // BN,),
        in_specs=[_row_spec(), _row_spec(), _deg_spec(), _deg_spec(),
                  _row_spec(), _w_spec(), _w_spec(), _b_spec()],
        out_specs=_row_spec(),
        out_shape=jax.ShapeDtypeStruct((N, D), jnp.float32),
    )(a0, a1, d0, d1, x, wl, wr, b)


def _tc2(m0, m1, d0, d1, x1, wl, wr, b, wa, wb, bl):
    return pl.pallas_call(
        _tc2_body,
        grid=(N // BN,),
        in_specs=[_row_spec(), _row_spec(), _deg_spec(), _deg_spec(),
                  _row_spec(), _w_spec(), _w_spec(), _b_spec(),
                  _w_spec(), _w_spec(), _b_spec()],
        out_specs=_row_spec(),
        out_shape=jax.ShapeDtypeStruct((N, D), jnp.float32),
    )(m0, m1, d0, d1, x1, wl, wr, b, wa, wb, bl)


def kernel(x, edge_index, W1_l, b1_l, W1_r, W2_l, b2_l, W2_r, W_lin, b_lin):
    src = edge_index[0].astype(jnp.int32)
    dst = edge_index[1].astype(jnp.int32)

    sums, deg = _segsum_deg(x, src, dst)
    a0, a1 = sums[:N], sums[N:]
    d0, d1 = deg[:N, None], deg[N:, None]

    x1 = _tc1(a0, a1, d0, d1, x, W1_l, W1_r, b1_l[None, :])

    sums2, = _segsum(x1, src, dst)
    m0, m1 = sums2[:N], sums2[N:]

    out = _tc2(m0, m1, d0, d1, x1, W2_l, W2_r, b2_l[None, :],
               W_lin[:, :D], W_lin[:, D:], b_lin[None, :])
    return out


# speedup vs baseline: 10.0279x; 1.5107x over previous
import functools

import jax
import jax.numpy as jnp
from jax import lax
from jax.experimental import pallas as pl
from jax.experimental.pallas import tpu as pltpu
from jax.experimental.pallas import tpu_sc as plsc

N = 10000
E = 320000
D = 128

K = 128
NC = 2
NS = 16
NW = NC * NS
CHUNKS = E // K
ITERS = (CHUNKS + NW - 1) // NW
ZR = 40

def _make_segsum(with_deg):
    mesh = plsc.VectorSubcoreMesh(core_axis_name="c", subcore_axis_name="s")
    out_type = [jax.ShapeDtypeStruct((NC * N, D), jnp.float32)]
    if with_deg:
        out_type.append(jax.ShapeDtypeStruct((NC * N,), jnp.float32))
    scratch = [
        pltpu.VMEM((K,), jnp.int32),
        pltpu.VMEM((K,), jnp.int32),
        pltpu.VMEM((K,), jnp.int32),
        pltpu.VMEM((K,), jnp.int32),
        pltpu.VMEM((K, D), jnp.float32),
        pltpu.VMEM((K, D), jnp.float32),
        pltpu.VMEM((K,), jnp.float32),
        pltpu.VMEM((ZR, D), jnp.float32),
        pltpu.VMEM((640,), jnp.float32),
        pltpu.VMEM_SHARED((N, D), jnp.float32),
        pltpu.VMEM_SHARED((N,), jnp.float32),
        pltpu.SemaphoreType.DMA,
        pltpu.SemaphoreType.DMA,
    ]

    def body(x_hbm, src_hbm, dst_hbm, *refs):
        if with_deg:
            out, deg_out = refs[0], refs[1]
            rest = refs[2:]
        else:
            out = refs[0]
            rest = refs[1:]
        (srcv0, srcv1, dstv0, dstv1, rows0, rows1, ones, zbuf, zdeg,
         acc, dacc, semg0, semg1) = rest
        bufs = ((srcv0, dstv0, rows0, semg0), (srcv1, dstv1, rows1, semg1))

        cid = lax.axis_index("c")
        sid = lax.axis_index("s")
        w = sid * NC + cid

        _zeros16 = jnp.zeros((16,), jnp.float32)
        _ones16 = jnp.ones((16,), jnp.float32)

        for r in range(ZR):
            for c in range(8):
                zbuf[r, pl.ds(c * 16, 16)] = _zeros16
        for i in range(640 // 16):
            zdeg[pl.ds(i * 16, 16)] = _zeros16
        if with_deg:
            for i in range(K // 16):
                ones[pl.ds(i * 16, 16)] = _ones16

        @pl.when(sid < 15)
        def _():
            def zrow(i, carry):
                pltpu.sync_copy(zbuf, acc.at[pl.ds(sid * 640 + i * ZR, ZR)])
                return carry
            lax.fori_loop(0, 16, zrow, 0)
            if with_deg:
                pltpu.sync_copy(zdeg, dacc.at[pl.ds(sid * 640, 640)])

        @pl.when(sid == 15)
        def _():
            def zrow(i, carry):
                pltpu.sync_copy(zbuf, acc.at[pl.ds(9600 + i * ZR, ZR)])
                return carry
            lax.fori_loop(0, 10, zrow, 0)
            if with_deg:
                pltpu.sync_copy(zdeg.at[pl.ds(0, 400)], dacc.at[pl.ds(9600, 400)])

        plsc.subcore_barrier()

        def slot(j_static_parity, j, carry):
            sv, dv, rw, sg = bufs[j_static_parity]
            pv, pd, pr, pg = bufs[1 - j_static_parity]
            chunk = w + j * NW

            @pl.when(chunk < CHUNKS)
            def _():
                base = chunk * K
                pltpu.sync_copy(src_hbm.at[pl.ds(base, K)], sv)
                pltpu.sync_copy(dst_hbm.at[pl.ds(base, K)], dv)
                pltpu.async_copy(x_hbm.at[sv], rw, sg)

            @pl.when((j >= 1) & (chunk - NW < CHUNKS))
            def _():
                pltpu.make_async_copy(x_hbm.at[pv], pr, pg).wait()
                pltpu.sync_copy(pr, acc.at[pd], add=True)
                if with_deg:
                    pltpu.sync_copy(ones, dacc.at[pd], add=True)

            return carry

        def step(it, carry):
            carry = slot(0, 2 * it, carry)
            carry = slot(1, 2 * it + 1, carry)
            return carry

        lax.fori_loop(0, (ITERS + 2) // 2, step, 0)

        plsc.subcore_barrier()

        @pl.when(sid < 15)
        def _():
            pltpu.sync_copy(acc.at[pl.ds(sid * 640, 640)],
                            out.at[pl.ds(cid * N + sid * 640, 640)])
            if with_deg:
                pltpu.sync_copy(dacc.at[pl.ds(sid * 640, 640)], zdeg)
                pltpu.sync_copy(zdeg,
                                deg_out.at[pl.ds(cid * N + sid * 640, 640)])

        @pl.when(sid == 15)
        def _():
            pltpu.sync_copy(acc.at[pl.ds(9600, 400)],
                            out.at[pl.ds(cid * N + 9600, 400)])
            if with_deg:
                pltpu.sync_copy(dacc.at[pl.ds(9600, 400)],
                                zdeg.at[pl.ds(0, 400)])
                pltpu.sync_copy(zdeg.at[pl.ds(0, 400)],
                                deg_out.at[pl.ds(cid * N + 9600, 400)])

    return pl.kernel(body, mesh=mesh, out_type=out_type, scratch_types=scratch)


_segsum_deg = _make_segsum(True)
_segsum = _make_segsum(False)


_CD = (((1,), (1,)), ((), ()))


def _tc1_body(a0, a1, d0, d1, x, wl, wr, b, o):
    deg = jnp.maximum(d0[...] + d1[...], 1.0)
    mean = (a0[...] + a1[...]) / deg
    y = lax.dot_general(mean, wl[...], _CD, preferred_element_type=jnp.float32)
    y = y + lax.dot_general(x[...], wr[...], _CD, preferred_element_type=jnp.float32)
    o[...] = jnp.maximum(y + b[...], 0.0)


def _tc2_body(m0, m1, d0, d1, x1, wl, wr, b, wa, wb, bl, o):
    deg = jnp.maximum(d0[...] + d1[...], 1.0)
    mean = (m0[...] + m1[...]) / deg
    y = lax.dot_general(mean, wl[...], _CD, preferred_element_type=jnp.float32)
    y = y + lax.dot_general(x1[...], wr[...], _CD, preferred_element_type=jnp.float32)
    x2 = jnp.maximum(y + b[...], 0.0)
    z = lax.dot_general(x1[...], wa[...], _CD, preferred_element_type=jnp.float32)
    z = z + lax.dot_general(x2, wb[...], _CD, preferred_element_type=jnp.float32)
    o[...] = z + bl[...]


BN = 1000


def _row_spec():
    return pl.BlockSpec((BN, D), lambda i: (i, 0))


def _deg_spec():
    return pl.BlockSpec((BN, 1), lambda i: (i, 0))


def _w_spec():
    return pl.BlockSpec((D, D), lambda i: (0, 0))


def _b_spec():
    return pl.BlockSpec((1, D), lambda i: (0, 0))


def _tc1(a0, a1, d0, d1, x, wl, wr, b):
    return pl.pallas_call(
        _tc1_body,
        grid=(N // BN,),
        in_specs=[_row_spec(), _row_spec(), _deg_spec(), _deg_spec(),
                  _row_spec(), _w_spec(), _w_spec(), _b_spec()],
        out_specs=_row_spec(),
        out_shape=jax.ShapeDtypeStruct((N, D), jnp.float32),
    )(a0, a1, d0, d1, x, wl, wr, b)


def _tc2(m0, m1, d0, d1, x1, wl, wr, b, wa, wb, bl):
    return pl.pallas_call(
        _tc2_body,
        grid=(N // BN,),
        in_specs=[_row_spec(), _row_spec(), _deg_spec(), _deg_spec(),
                  _row_spec(), _w_spec(), _w_spec(), _b_spec(),
                  _w_spec(), _w_spec(), _b_spec()],
        out_specs=_row_spec(),
        out_shape=jax.ShapeDtypeStruct((N, D), jnp.float32),
    )(m0, m1, d0, d1, x1, wl, wr, b, wa, wb, bl)


def kernel(x, edge_index, W1_l, b1_l, W1_r, W2_l, b2_l, W2_r, W_lin, b_lin):
    src = edge_index[0].astype(jnp.int32)
    dst = edge_index[1].astype(jnp.int32)

    sums, deg = _segsum_deg(x, src, dst)
    a0, a1 = sums[:N], sums[N:]
    d0, d1 = deg[:N, None], deg[N:, None]

    x1 = _tc1(a0, a1, d0, d1, x, W1_l, W1_r, b1_l[None, :])

    sums2, = _segsum(x1, src, dst)
    m0, m1 = sums2[:N], sums2[N:]

    out = _tc2(m0, m1, d0, d1, x1, W2_l, W2_r, b2_l[None, :],
               W_lin[:, :D], W_lin[:, D:], b_lin[None, :])
    return out


# speedup vs baseline: 10.5215x; 1.0492x over previous
import functools

import jax
import jax.numpy as jnp
from jax import lax
from jax.experimental import pallas as pl
from jax.experimental.pallas import tpu as pltpu
from jax.experimental.pallas import tpu_sc as plsc

N = 10000
E = 320000
D = 128

K = 128
NC = 2
NS = 16
NW = NC * NS
CHUNKS = E // K
ITERS = (CHUNKS + NW - 1) // NW
ZR = 40

def _make_segsum(with_deg):
    mesh = plsc.VectorSubcoreMesh(core_axis_name="c", subcore_axis_name="s")
    out_type = [jax.ShapeDtypeStruct((NC * N, D), jnp.float32)]
    if with_deg:
        out_type.append(jax.ShapeDtypeStruct((NC * N,), jnp.float32))
    scratch = [
        pltpu.VMEM((K,), jnp.int32),
        pltpu.VMEM((K,), jnp.int32),
        pltpu.VMEM((K,), jnp.int32),
        pltpu.VMEM((K,), jnp.int32),
        pltpu.VMEM((K, D), jnp.float32),
        pltpu.VMEM((K, D), jnp.float32),
        pltpu.VMEM((K,), jnp.float32),
        pltpu.VMEM((ZR, D), jnp.float32),
        pltpu.VMEM((640,), jnp.float32),
        pltpu.VMEM_SHARED((N, D), jnp.float32),
        pltpu.VMEM_SHARED((N,), jnp.float32),
        pltpu.SemaphoreType.DMA,
        pltpu.SemaphoreType.DMA,
        pltpu.SemaphoreType.DMA,
        pltpu.SemaphoreType.DMA,
    ]

    def body(x_hbm, src_hbm, dst_hbm, *refs):
        if with_deg:
            out, deg_out = refs[0], refs[1]
            rest = refs[2:]
        else:
            out = refs[0]
            rest = refs[1:]
        (srcv0, srcv1, dstv0, dstv1, rows0, rows1, ones, zbuf, zdeg,
         acc, dacc, semg0, semg1, sems0, sems1) = rest
        bufs = ((srcv0, dstv0, rows0, semg0, sems0),
                (srcv1, dstv1, rows1, semg1, sems1))

        cid = lax.axis_index("c")
        sid = lax.axis_index("s")
        w = sid * NC + cid

        _zeros16 = jnp.zeros((16,), jnp.float32)
        _ones16 = jnp.ones((16,), jnp.float32)

        for r in range(ZR):
            for c in range(8):
                zbuf[r, pl.ds(c * 16, 16)] = _zeros16
        for i in range(640 // 16):
            zdeg[pl.ds(i * 16, 16)] = _zeros16
        if with_deg:
            for i in range(K // 16):
                ones[pl.ds(i * 16, 16)] = _ones16

        @pl.when(sid < 15)
        def _():
            def zrow(i, carry):
                pltpu.sync_copy(zbuf, acc.at[pl.ds(sid * 640 + i * ZR, ZR)])
                return carry
            lax.fori_loop(0, 16, zrow, 0)
            if with_deg:
                pltpu.sync_copy(zdeg, dacc.at[pl.ds(sid * 640, 640)])

        @pl.when(sid == 15)
        def _():
            def zrow(i, carry):
                pltpu.sync_copy(zbuf, acc.at[pl.ds(9600 + i * ZR, ZR)])
                return carry
            lax.fori_loop(0, 10, zrow, 0)
            if with_deg:
                pltpu.sync_copy(zdeg.at[pl.ds(0, 400)], dacc.at[pl.ds(9600, 400)])

        plsc.subcore_barrier()

        def slot(parity, j, carry):
            sv, dv, rw, sg, ss = bufs[parity]
            pv, pd, pr, pg, ps = bufs[1 - parity]
            chunk = w + j * NW

            @pl.when((j >= 2) & (chunk - 2 * NW < CHUNKS))
            def _():
                pltpu.make_async_copy(rw, acc.at[dv], ss).wait()

            @pl.when(chunk < CHUNKS)
            def _():
                base = chunk * K
                pltpu.sync_copy(src_hbm.at[pl.ds(base, K)], sv)
                pltpu.sync_copy(dst_hbm.at[pl.ds(base, K)], dv)
                pltpu.async_copy(x_hbm.at[sv], rw, sg)

            @pl.when((j >= 1) & (chunk - NW < CHUNKS))
            def _():
                pltpu.make_async_copy(x_hbm.at[pv], pr, pg).wait()
                pltpu.async_copy(pr, acc.at[pd], ps, add=True)
                if with_deg:
                    pltpu.sync_copy(ones, dacc.at[pd], add=True)

            return carry

        def step(it, carry):
            carry = slot(0, 2 * it, carry)
            carry = slot(1, 2 * it + 1, carry)
            return carry

        lax.fori_loop(0, (ITERS + 3) // 2 + 1, step, 0)

        plsc.subcore_barrier()

        @pl.when(sid < 15)
        def _():
            pltpu.sync_copy(acc.at[pl.ds(sid * 640, 640)],
                            out.at[pl.ds(cid * N + sid * 640, 640)])
            if with_deg:
                pltpu.sync_copy(dacc.at[pl.ds(sid * 640, 640)], zdeg)
                pltpu.sync_copy(zdeg,
                                deg_out.at[pl.ds(cid * N + sid * 640, 640)])

        @pl.when(sid == 15)
        def _():
            pltpu.sync_copy(acc.at[pl.ds(9600, 400)],
                            out.at[pl.ds(cid * N + 9600, 400)])
            if with_deg:
                pltpu.sync_copy(dacc.at[pl.ds(9600, 400)],
                                zdeg.at[pl.ds(0, 400)])
                pltpu.sync_copy(zdeg.at[pl.ds(0, 400)],
                                deg_out.at[pl.ds(cid * N + 9600, 400)])

    return pl.kernel(body, mesh=mesh, out_type=out_type, scratch_types=scratch)


_segsum_deg = _make_segsum(True)
_segsum = _make_segsum(False)


_CD = (((1,), (1,)), ((), ()))


def _tc1_body(a0, a1, d0, d1, x, wl, wr, b, o):
    deg = jnp.maximum(d0[...] + d1[...], 1.0)
    mean = (a0[...] + a1[...]) / deg
    y = lax.dot_general(mean, wl[...], _CD, preferred_element_type=jnp.float32)
    y = y + lax.dot_general(x[...], wr[...], _CD, preferred_element_type=jnp.float32)
    o[...] = jnp.maximum(y + b[...], 0.0)


def _tc2_body(m0, m1, d0, d1, x1, wl, wr, b, wa, wb, bl, o):
    deg = jnp.maximum(d0[...] + d1[...], 1.0)
    mean = (m0[...] + m1[...]) / deg
    y = lax.dot_general(mean, wl[...], _CD, preferred_element_type=jnp.float32)
    y = y + lax.dot_general(x1[...], wr[...], _CD, preferred_element_type=jnp.float32)
    x2 = jnp.maximum(y + b[...], 0.0)
    z = lax.dot_general(x1[...], wa[...], _CD, preferred_element_type=jnp.float32)
    z = z + lax.dot_general(x2, wb[...], _CD, preferred_element_type=jnp.float32)
    o[...] = z + bl[...]


BN = 1000
NB = N // BN


def _row_spec(off=0):
    return pl.BlockSpec((BN, D), lambda i, o=off: (i + o, 0))


def _deg_spec(off=0):
    return pl.BlockSpec((BN, 1), lambda i, o=off: (i + o, 0))


def _w_spec():
    return pl.BlockSpec((D, D), lambda i: (0, 0))


def _b_spec():
    return pl.BlockSpec((1, D), lambda i: (0, 0))


def _tc1(sums, deg2, x, wl, wr, b):
    return pl.pallas_call(
        _tc1_body,
        grid=(NB,),
        in_specs=[_row_spec(), _row_spec(NB), _deg_spec(), _deg_spec(NB),
                  _row_spec(), _w_spec(), _w_spec(), _b_spec()],
        out_specs=pl.BlockSpec((BN, D), lambda i: (i, 0)),
        out_shape=jax.ShapeDtypeStruct((N, D), jnp.float32),
    )(sums, sums, deg2, deg2, x, wl, wr, b)


def _tc2(sums2, deg2, x1, wl, wr, b, wa, wb, bl):
    return pl.pallas_call(
        _tc2_body,
        grid=(NB,),
        in_specs=[_row_spec(), _row_spec(NB), _deg_spec(), _deg_spec(NB),
                  _row_spec(), _w_spec(), _w_spec(), _b_spec(),
                  _w_spec(), _w_spec(), _b_spec()],
        out_specs=pl.BlockSpec((BN, D), lambda i: (i, 0)),
        out_shape=jax.ShapeDtypeStruct((N, D), jnp.float32),
    )(sums2, sums2, deg2, deg2, x1, wl, wr, b, wa, wb, bl)


def kernel(x, edge_index, W1_l, b1_l, W1_r, W2_l, b2_l, W2_r, W_lin, b_lin):
    src = edge_index[0].astype(jnp.int32)
    dst = edge_index[1].astype(jnp.int32)

    sums, deg = _segsum_deg(x, src, dst)
    deg2 = deg[:, None]

    x1 = _tc1(sums, deg2, x, W1_l, W1_r, b1_l[None, :])

    sums2, = _segsum(x1, src, dst)

    out = _tc2(sums2, deg2, x1, W2_l, W2_r, b2_l[None, :],
               W_lin[:, :D], W_lin[:, D:], b_lin[None, :])
    return out


# speedup vs baseline: 12.9511x; 1.2309x over previous
import functools

import jax
import jax.numpy as jnp
from jax import lax
from jax.experimental import pallas as pl
from jax.experimental.pallas import tpu as pltpu
from jax.experimental.pallas import tpu_sc as plsc

N = 10000
E = 320000
D = 128

K = 128
NC = 2
NS = 16
NW = NC * NS
WROWS = 2560
EP = WROWS * K
WPT = WROWS // NW
SW = 8
NA = N + 8
ZR = 40

def _make_segsum(with_deg):
    mesh = plsc.VectorSubcoreMesh(core_axis_name="c", subcore_axis_name="s")
    out_type = [jax.ShapeDtypeStruct((NC * N, D), jnp.float32)]
    if with_deg:
        out_type.append(jax.ShapeDtypeStruct((NC * N,), jnp.float32))
    scratch = [
        pltpu.VMEM((SW, K), jnp.int32),
        pltpu.VMEM((SW, K), jnp.int32),
        pltpu.VMEM((SW, K), jnp.int32),
        pltpu.VMEM((SW, K), jnp.int32),
        pltpu.VMEM((K, D), jnp.float32),
        pltpu.VMEM((K, D), jnp.float32),
        pltpu.VMEM((K,), jnp.float32),
        pltpu.VMEM((ZR, D), jnp.float32),
        pltpu.VMEM((640,), jnp.float32),
        pltpu.VMEM_SHARED((NA, D), jnp.float32),
        pltpu.VMEM_SHARED((NA,), jnp.float32),
        pltpu.SemaphoreType.DMA,
        pltpu.SemaphoreType.DMA,
        pltpu.SemaphoreType.DMA,
        pltpu.SemaphoreType.DMA,
    ]

    def body(x_hbm, src_hbm, dst_hbm, *refs):
        if with_deg:
            out, deg_out = refs[0], refs[1]
            rest = refs[2:]
        else:
            out = refs[0]
            rest = refs[1:]
        (srcA, srcB, dstA, dstB, rows0, rows1, ones, zbuf, zdeg,
         acc, dacc, semg0, semg1, sems0, sems1) = rest
        rowbufs = (rows0, rows1)
        semg = (semg0, semg1)
        sems = (sems0, sems1)

        cid = lax.axis_index("c")
        sid = lax.axis_index("s")
        w = sid * NC + cid

        _zeros16 = jnp.zeros((16,), jnp.float32)
        _ones16 = jnp.ones((16,), jnp.float32)

        for r in range(ZR):
            for c in range(8):
                zbuf[r, pl.ds(c * 16, 16)] = _zeros16
        for i in range(640 // 16):
            zdeg[pl.ds(i * 16, 16)] = _zeros16
        if with_deg:
            for i in range(K // 16):
                ones[pl.ds(i * 16, 16)] = _ones16

        @pl.when(sid < 15)
        def _():
            def zrow(i, carry):
                pltpu.sync_copy(zbuf, acc.at[pl.ds(sid * 640 + i * ZR, ZR)])
                return carry
            lax.fori_loop(0, 16, zrow, 0)
            if with_deg:
                pltpu.sync_copy(zdeg, dacc.at[pl.ds(sid * 640, 640)])

        @pl.when(sid == 15)
        def _():
            def zrow(i, carry):
                pltpu.sync_copy(zbuf, acc.at[pl.ds(9600 + i * ZR, ZR)])
                return carry
            lax.fori_loop(0, 10, zrow, 0)
            if with_deg:
                pltpu.sync_copy(zdeg.at[pl.ds(0, 400)], dacc.at[pl.ds(9600, 400)])

        plsc.subcore_barrier()

        start = w * WPT

        def drain_scatter(p):
            pltpu.make_async_copy(rowbufs[p], acc.at[dstA.at[0]],
                                  sems[p]).wait()

        def do_scatter(p, dref):
            pltpu.make_async_copy(x_hbm.at[srcA.at[0]], rowbufs[p],
                                  semg[p]).wait()
            pltpu.async_copy(rowbufs[p], acc.at[dref], sems[p], add=True)
            if with_deg:
                pltpu.sync_copy(ones, dacc.at[dref], add=True)

        def do_superstep(t, is_b):
            s = 2 * t + (1 if is_b else 0)
            cur_src, cur_dst = (srcB, dstB) if is_b else (srcA, dstA)
            prv_dst = dstA if is_b else dstB
            row0 = start + SW * s
            pltpu.sync_copy(src_hbm.at[pl.ds(row0, SW)], cur_src)
            pltpu.sync_copy(dst_hbm.at[pl.ds(row0, SW)], cur_dst)
            for jj in range(SW):
                p = jj % 2
                if (not is_b) and jj < 2:
                    @pl.when(t > 0)
                    def _(p=p):
                        drain_scatter(p)
                else:
                    drain_scatter(p)
                pltpu.async_copy(x_hbm.at[cur_src.at[jj]], rowbufs[p],
                                 semg[p])
                dref = cur_dst.at[jj - 1] if jj >= 1 else prv_dst.at[SW - 1]
                if (not is_b) and jj == 0:
                    @pl.when(t > 0)
                    def _(p=p, dref=dref):
                        do_scatter(1 - p, dref)
                else:
                    do_scatter(1 - p, dref)

        def pair(t, carry):
            do_superstep(t, False)
            do_superstep(t, True)
            return carry

        lax.fori_loop(0, WPT // SW // 2, pair, 0)

        do_scatter(1, dstB.at[SW - 1])
        drain_scatter(0)
        drain_scatter(1)

        plsc.subcore_barrier()

        @pl.when(sid < 15)
        def _():
            pltpu.sync_copy(acc.at[pl.ds(sid * 640, 640)],
                            out.at[pl.ds(cid * N + sid * 640, 640)])
            if with_deg:
                pltpu.sync_copy(dacc.at[pl.ds(sid * 640, 640)], zdeg)
                pltpu.sync_copy(zdeg,
                                deg_out.at[pl.ds(cid * N + sid * 640, 640)])

        @pl.when(sid == 15)
        def _():
            pltpu.sync_copy(acc.at[pl.ds(9600, 400)],
                            out.at[pl.ds(cid * N + 9600, 400)])
            if with_deg:
                pltpu.sync_copy(dacc.at[pl.ds(9600, 400)],
                                zdeg.at[pl.ds(0, 400)])
                pltpu.sync_copy(zdeg.at[pl.ds(0, 400)],
                                deg_out.at[pl.ds(cid * N + 9600, 400)])

    return pl.kernel(body, mesh=mesh, out_type=out_type, scratch_types=scratch)


_segsum_deg = _make_segsum(True)
_segsum = _make_segsum(False)


_CD = (((1,), (1,)), ((), ()))


def _tc1_body(a0, a1, d0, d1, x, wl, wr, b, o):
    deg = jnp.maximum(d0[...] + d1[...], 1.0)
    mean = (a0[...] + a1[...]) / deg
    y = lax.dot_general(mean, wl[...], _CD, preferred_element_type=jnp.float32)
    y = y + lax.dot_general(x[...], wr[...], _CD, preferred_element_type=jnp.float32)
    o[...] = jnp.maximum(y + b[...], 0.0)


def _tc2_body(m0, m1, d0, d1, x1, wl, wr, b, wa, wb, bl, o):
    deg = jnp.maximum(d0[...] + d1[...], 1.0)
    mean = (m0[...] + m1[...]) / deg
    y = lax.dot_general(mean, wl[...], _CD, preferred_element_type=jnp.float32)
    y = y + lax.dot_general(x1[...], wr[...], _CD, preferred_element_type=jnp.float32)
    x2 = jnp.maximum(y + b[...], 0.0)
    z = lax.dot_general(x1[...], wa[...], _CD, preferred_element_type=jnp.float32)
    z = z + lax.dot_general(x2, wb[...], _CD, preferred_element_type=jnp.float32)
    o[...] = z + bl[...]


BN = 1000
NB = N // BN


def _row_spec(off=0):
    return pl.BlockSpec((BN, D), lambda i, o=off: (i + o, 0))


def _deg_spec(off=0):
    return pl.BlockSpec((BN, 1), lambda i, o=off: (i + o, 0))


def _w_spec():
    return pl.BlockSpec((D, D), lambda i: (0, 0))


def _b_spec():
    return pl.BlockSpec((1, D), lambda i: (0, 0))


def _tc1(sums, deg2, x, wl, wr, b):
    return pl.pallas_call(
        _tc1_body,
        grid=(NB,),
        in_specs=[_row_spec(), _row_spec(NB), _deg_spec(), _deg_spec(NB),
                  _row_spec(), _w_spec(), _w_spec(), _b_spec()],
        out_specs=pl.BlockSpec((BN, D), lambda i: (i, 0)),
        out_shape=jax.ShapeDtypeStruct((N, D), jnp.float32),
    )(sums, sums, deg2, deg2, x, wl, wr, b)


def _tc2(sums2, deg2, x1, wl, wr, b, wa, wb, bl):
    return pl.pallas_call(
        _tc2_body,
        grid=(NB,),
        in_specs=[_row_spec(), _row_spec(NB), _deg_spec(), _deg_spec(NB),
                  _row_spec(), _w_spec(), _w_spec(), _b_spec(),
                  _w_spec(), _w_spec(), _b_spec()],
        out_specs=pl.BlockSpec((BN, D), lambda i: (i, 0)),
        out_shape=jax.ShapeDtypeStruct((N, D), jnp.float32),
    )(sums2, sums2, deg2, deg2, x1, wl, wr, b, wa, wb, bl)


def kernel(x, edge_index, W1_l, b1_l, W1_r, W2_l, b2_l, W2_r, W_lin, b_lin):
    src = edge_index[0].astype(jnp.int32)
    dst = edge_index[1].astype(jnp.int32)
    pidx = jnp.arange(EP - E, dtype=jnp.int32)
    src = jnp.concatenate([src, pidx % N]).reshape(WROWS, K)
    dst = jnp.concatenate([dst, N + (pidx % 8)]).reshape(WROWS, K)

    sums, deg = _segsum_deg(x, src, dst)
    deg2 = deg[:, None]

    x1 = _tc1(sums, deg2, x, W1_l, W1_r, b1_l[None, :])

    sums2, = _segsum(x1, src, dst)

    out = _tc2(sums2, deg2, x1, W2_l, W2_r, b2_l[None, :],
               W_lin[:, :D], W_lin[:, D:], b_lin[None, :])
    return out


# speedup vs baseline: 13.3946x; 1.0342x over previous
import functools

import jax
import jax.numpy as jnp
from jax import lax
from jax.experimental import pallas as pl
from jax.experimental.pallas import tpu as pltpu
from jax.experimental.pallas import tpu_sc as plsc

N = 10000
E = 320000
D = 128

K = 128
NC = 2
NS = 16
NW = NC * NS
WROWS = 2560
EP = WROWS * K
WPT = WROWS // NW
SW = 8
NA = N + 8
ZR = 40

def _make_segsum(with_deg):
    mesh = plsc.VectorSubcoreMesh(core_axis_name="c", subcore_axis_name="s")
    out_type = [jax.ShapeDtypeStruct((NC * N, D), jnp.float32)]
    if with_deg:
        out_type.append(jax.ShapeDtypeStruct((NC * N,), jnp.float32))
    scratch = [
        pltpu.VMEM((SW, K), jnp.int32),
        pltpu.VMEM((SW, K), jnp.int32),
        pltpu.VMEM((SW, K), jnp.int32),
        pltpu.VMEM((SW, K), jnp.int32),
        pltpu.VMEM((K, D), jnp.float32),
        pltpu.VMEM((K, D), jnp.float32),
        pltpu.VMEM((K,), jnp.float32),
        pltpu.VMEM((ZR, D), jnp.float32),
        pltpu.VMEM((640,), jnp.float32),
        pltpu.VMEM_SHARED((NA, D), jnp.float32),
        pltpu.VMEM_SHARED((NA,), jnp.float32),
        pltpu.SemaphoreType.DMA,
        pltpu.SemaphoreType.DMA,
        pltpu.SemaphoreType.DMA,
        pltpu.SemaphoreType.DMA,
    ]

    def body(x_hbm, ei_hbm, *refs):
        if with_deg:
            out, deg_out = refs[0], refs[1]
            rest = refs[2:]
        else:
            out = refs[0]
            rest = refs[1:]
        (srcA, srcB, dstA, dstB, rows0, rows1, ones, zbuf, zdeg,
         acc, dacc, semg0, semg1, sems0, sems1) = rest
        rowbufs = (rows0, rows1)
        semg = (semg0, semg1)
        sems = (sems0, sems1)

        cid = lax.axis_index("c")
        sid = lax.axis_index("s")
        w = sid * NC + cid

        _zeros16 = jnp.zeros((16,), jnp.float32)
        _ones16 = jnp.ones((16,), jnp.float32)

        for r in range(ZR):
            for c in range(8):
                zbuf[r, pl.ds(c * 16, 16)] = _zeros16
        for i in range(640 // 16):
            zdeg[pl.ds(i * 16, 16)] = _zeros16
        if with_deg:
            for i in range(K // 16):
                ones[pl.ds(i * 16, 16)] = _ones16

        @pl.when(sid < 15)
        def _():
            def zrow(i, carry):
                pltpu.sync_copy(zbuf, acc.at[pl.ds(sid * 640 + i * ZR, ZR)])
                return carry
            lax.fori_loop(0, 16, zrow, 0)
            if with_deg:
                pltpu.sync_copy(zdeg, dacc.at[pl.ds(sid * 640, 640)])

        @pl.when(sid == 15)
        def _():
            def zrow(i, carry):
                pltpu.sync_copy(zbuf, acc.at[pl.ds(9600 + i * ZR, ZR)])
                return carry
            lax.fori_loop(0, 10, zrow, 0)
            if with_deg:
                pltpu.sync_copy(zdeg.at[pl.ds(0, 400)], dacc.at[pl.ds(9600, 400)])

        plsc.subcore_barrier()

        start = w * WPT

        def drain_scatter(p):
            pltpu.make_async_copy(rowbufs[p], acc.at[dstA.at[0]],
                                  sems[p]).wait()

        def do_scatter(p, dref):
            pltpu.make_async_copy(x_hbm.at[srcA.at[0]], rowbufs[p],
                                  semg[p]).wait()
            pltpu.async_copy(rowbufs[p], acc.at[dref], sems[p], add=True)
            if with_deg:
                pltpu.sync_copy(ones, dacc.at[dref], add=True)

        def do_superstep(t, is_b):
            s = 2 * t + (1 if is_b else 0)
            cur_src, cur_dst = (srcB, dstB) if is_b else (srcA, dstA)
            prv_dst = dstA if is_b else dstB
            row0 = start + SW * s
            pltpu.sync_copy(ei_hbm.at[0, pl.ds(row0, SW)], cur_src)
            pltpu.sync_copy(ei_hbm.at[1, pl.ds(row0, SW)], cur_dst)
            for jj in range(SW):
                p = jj % 2
                if (not is_b) and jj < 2:
                    @pl.when(t > 0)
                    def _(p=p):
                        drain_scatter(p)
                else:
                    drain_scatter(p)
                pltpu.async_copy(x_hbm.at[cur_src.at[jj]], rowbufs[p],
                                 semg[p])
                dref = cur_dst.at[jj - 1] if jj >= 1 else prv_dst.at[SW - 1]
                if (not is_b) and jj == 0:
                    @pl.when(t > 0)
                    def _(p=p, dref=dref):
                        do_scatter(1 - p, dref)
                else:
                    do_scatter(1 - p, dref)

        def pair(t, carry):
            do_superstep(t, False)
            do_superstep(t, True)
            return carry

        lax.fori_loop(0, WPT // SW // 2, pair, 0)

        do_scatter(1, dstB.at[SW - 1])
        drain_scatter(0)
        drain_scatter(1)

        plsc.subcore_barrier()

        @pl.when(sid < 15)
        def _():
            pltpu.sync_copy(acc.at[pl.ds(sid * 640, 640)],
                            out.at[pl.ds(cid * N + sid * 640, 640)])
            if with_deg:
                pltpu.sync_copy(dacc.at[pl.ds(sid * 640, 640)], zdeg)
                pltpu.sync_copy(zdeg,
                                deg_out.at[pl.ds(cid * N + sid * 640, 640)])

        @pl.when(sid == 15)
        def _():
            pltpu.sync_copy(acc.at[pl.ds(9600, 400)],
                            out.at[pl.ds(cid * N + 9600, 400)])
            if with_deg:
                pltpu.sync_copy(dacc.at[pl.ds(9600, 400)],
                                zdeg.at[pl.ds(0, 400)])
                pltpu.sync_copy(zdeg.at[pl.ds(0, 400)],
                                deg_out.at[pl.ds(cid * N + 9600, 400)])

    return pl.kernel(body, mesh=mesh, out_type=out_type, scratch_types=scratch)


_segsum_deg = _make_segsum(True)
_segsum = _make_segsum(False)


_CD = (((1,), (1,)), ((), ()))


def _tc1_body(a0, a1, d0, d1, x, wl, wr, b, o):
    deg = jnp.maximum(d0[...] + d1[...], 1.0)
    mean = (a0[...] + a1[...]) / deg
    y = lax.dot_general(mean, wl[...], _CD, preferred_element_type=jnp.float32)
    y = y + lax.dot_general(x[...], wr[...], _CD, preferred_element_type=jnp.float32)
    o[...] = jnp.maximum(y + b[...], 0.0)


def _tc2_body(m0, m1, d0, d1, x1, wl, wr, b, wa, wb, bl, o):
    deg = jnp.maximum(d0[...] + d1[...], 1.0)
    mean = (m0[...] + m1[...]) / deg
    y = lax.dot_general(mean, wl[...], _CD, preferred_element_type=jnp.float32)
    y = y + lax.dot_general(x1[...], wr[...], _CD, preferred_element_type=jnp.float32)
    x2 = jnp.maximum(y + b[...], 0.0)
    z = lax.dot_general(x1[...], wa[...], _CD, preferred_element_type=jnp.float32)
    z = z + lax.dot_general(x2, wb[...], _CD, preferred_element_type=jnp.float32)
    o[...] = z + bl[...]


BN = 2000
NB = N // BN


def _row_spec(off=0):
    return pl.BlockSpec((BN, D), lambda i, o=off: (i + o, 0))


def _deg_spec(off=0):
    return pl.BlockSpec((BN, 1), lambda i, o=off: (i + o, 0))


def _w_spec():
    return pl.BlockSpec((D, D), lambda i: (0, 0))


def _b_spec():
    return pl.BlockSpec((1, D), lambda i: (0, 0))


def _tc1(sums, deg2, x, wl, wr, b):
    return pl.pallas_call(
        _tc1_body,
        grid=(NB,),
        in_specs=[_row_spec(), _row_spec(NB), _deg_spec(), _deg_spec(NB),
                  _row_spec(), _w_spec(), _w_spec(), _b_spec()],
        out_specs=pl.BlockSpec((BN, D), lambda i: (i, 0)),
        out_shape=jax.ShapeDtypeStruct((N, D), jnp.float32),
    )(sums, sums, deg2, deg2, x, wl, wr, b)


def _tc2(sums2, deg2, x1, wl, wr, b, wa, wb, bl):
    return pl.pallas_call(
        _tc2_body,
        grid=(NB,),
        in_specs=[_row_spec(), _row_spec(NB), _deg_spec(), _deg_spec(NB),
                  _row_spec(), _w_spec(), _w_spec(), _b_spec(),
                  _w_spec(), _w_spec(), _b_spec()],
        out_specs=pl.BlockSpec((BN, D), lambda i: (i, 0)),
        out_shape=jax.ShapeDtypeStruct((N, D), jnp.float32),
    )(sums2, sums2, deg2, deg2, x1, wl, wr, b, wa, wb, bl)


def kernel(x, edge_index, W1_l, b1_l, W1_r, W2_l, b2_l, W2_r, W_lin, b_lin):
    pidx = jnp.arange(EP - E, dtype=jnp.int32)
    pad = jnp.stack([pidx % N, N + (pidx % 8)])
    ei = jnp.concatenate([edge_index.astype(jnp.int32), pad],
                         axis=1).reshape(2, WROWS, K)

    sums, deg = _segsum_deg(x, ei)
    deg2 = deg[:, None]

    x1 = _tc1(sums, deg2, x, W1_l, W1_r, b1_l[None, :])

    sums2, = _segsum(x1, ei)

    out = _tc2(sums2, deg2, x1, W2_l, W2_r, b2_l[None, :],
               W_lin[:, :D], W_lin[:, D:], b_lin[None, :])
    return out


# speedup vs baseline: 13.9694x; 1.0429x over previous
import functools

import jax
import jax.numpy as jnp
from jax import lax
from jax.experimental import pallas as pl
from jax.experimental.pallas import tpu as pltpu
from jax.experimental.pallas import tpu_sc as plsc

N = 10000
E = 320000
D = 128

K = 128
NC = 2
NS = 16
NW = NC * NS
WROWS = 2560
EP = WROWS * K
WPT = WROWS // NW
SW = 8
NA = N + 8
ZR = 40

def _make_segsum(with_deg):
    mesh = plsc.VectorSubcoreMesh(core_axis_name="c", subcore_axis_name="s")
    out_type = [jax.ShapeDtypeStruct((NC * N, D), jnp.float32)]
    if with_deg:
        out_type.append(jax.ShapeDtypeStruct((NC * N,), jnp.float32))
    scratch = [
        pltpu.VMEM((SW, K), jnp.int32),
        pltpu.VMEM((SW, K), jnp.int32),
        pltpu.VMEM((SW, K), jnp.int32),
        pltpu.VMEM((SW, K), jnp.int32),
        pltpu.VMEM((K, D), jnp.float32),
        pltpu.VMEM((K, D), jnp.float32),
        pltpu.VMEM((K,), jnp.float32),
        pltpu.VMEM((ZR, D), jnp.float32),
        pltpu.VMEM((640,), jnp.float32),
        pltpu.VMEM_SHARED((NA, D), jnp.float32),
        pltpu.VMEM_SHARED((NA,), jnp.float32),
        pltpu.SemaphoreType.DMA,
        pltpu.SemaphoreType.DMA,
        pltpu.SemaphoreType.DMA,
        pltpu.SemaphoreType.DMA,
        pltpu.SemaphoreType.DMA,
    ]

    def body(x_hbm, ei_hbm, *refs):
        if with_deg:
            out, deg_out = refs[0], refs[1]
            rest = refs[2:]
        else:
            out = refs[0]
            rest = refs[1:]
        (srcA, srcB, dstA, dstB, rows0, rows1, ones, zbuf,
         zdeg, acc, dacc, semg0, semg1, sems0, sems1, semi) = rest
        rowbufs = (rows0, rows1)
        semg = (semg0, semg1)
        sems = (sems0, sems1)

        cid = lax.axis_index("c")
        sid = lax.axis_index("s")
        w = sid * NC + cid

        _zeros16 = jnp.zeros((16,), jnp.float32)
        _ones16 = jnp.ones((16,), jnp.float32)

        for r in range(ZR):
            for c in range(8):
                zbuf[r, pl.ds(c * 16, 16)] = _zeros16
        for i in range(640 // 16):
            zdeg[pl.ds(i * 16, 16)] = _zeros16
        if with_deg:
            for i in range(K // 16):
                ones[pl.ds(i * 16, 16)] = _ones16

        @pl.when(sid < 15)
        def _():
            def zrow(i, carry):
                pltpu.sync_copy(zbuf, acc.at[pl.ds(sid * 640 + i * ZR, ZR)])
                return carry
            lax.fori_loop(0, 16, zrow, 0)
            if with_deg:
                pltpu.sync_copy(zdeg, dacc.at[pl.ds(sid * 640, 640)])

        @pl.when(sid == 15)
        def _():
            def zrow(i, carry):
                pltpu.sync_copy(zbuf, acc.at[pl.ds(9600 + i * ZR, ZR)])
                return carry
            lax.fori_loop(0, 10, zrow, 0)
            if with_deg:
                pltpu.sync_copy(zdeg.at[pl.ds(0, 400)], dacc.at[pl.ds(9600, 400)])

        plsc.subcore_barrier()

        start = w * WPT

        def drain_scatter(p):
            pltpu.make_async_copy(rowbufs[p], acc.at[dstA.at[0]],
                                  sems[p]).wait()

        def do_scatter(p, dref):
            pltpu.make_async_copy(x_hbm.at[srcA.at[0]], rowbufs[p],
                                  semg[p]).wait()
            pltpu.async_copy(rowbufs[p], acc.at[dref], sems[p], add=True)
            if with_deg:
                pltpu.sync_copy(ones, dacc.at[dref], add=True)

        def wait_idx(cur_src, cur_dst):
            pltpu.make_async_copy(ei_hbm.at[0, pl.ds(start, SW)],
                                  cur_src, semi).wait()
            pltpu.make_async_copy(ei_hbm.at[1, pl.ds(start, SW)],
                                  cur_dst, semi).wait()

        def do_superstep(t, is_b):
            s = 2 * t + (1 if is_b else 0)
            cur_src, cur_dst = (srcB, dstB) if is_b else (srcA, dstA)
            prv_dst = dstA if is_b else dstB
            nxt_src, nxt_dst = (srcA, dstA) if is_b else (srcB, dstB)

            if not is_b:
                @pl.when(t > 0)
                def _():
                    wait_idx(cur_src, cur_dst)
            else:
                wait_idx(cur_src, cur_dst)

            for jj in range(SW):
                p = jj % 2
                if (not is_b) and jj < 2:
                    @pl.when(t > 0)
                    def _(p=p):
                        drain_scatter(p)
                else:
                    drain_scatter(p)
                pltpu.async_copy(x_hbm.at[cur_src.at[jj]], rowbufs[p],
                                 semg[p])
                dref = cur_dst.at[jj - 1] if jj >= 1 else prv_dst.at[SW - 1]
                if (not is_b) and jj == 0:
                    @pl.when(t > 0)
                    def _(p=p, dref=dref):
                        do_scatter(1 - p, dref)
                else:
                    do_scatter(1 - p, dref)
                if jj == 1:
                    nrow0 = start + SW * (s + 1)
                    if not is_b:
                        pltpu.async_copy(ei_hbm.at[0, pl.ds(nrow0, SW)],
                                         nxt_src, semi)
                        pltpu.async_copy(ei_hbm.at[1, pl.ds(nrow0, SW)],
                                         nxt_dst, semi)
                    else:
                        @pl.when(t < WPT // SW // 2 - 1)
                        def _(nrow0=nrow0):
                            pltpu.async_copy(ei_hbm.at[0, pl.ds(nrow0, SW)],
                                             nxt_src, semi)
                            pltpu.async_copy(ei_hbm.at[1, pl.ds(nrow0, SW)],
                                             nxt_dst, semi)

        def pair(t, carry):
            do_superstep(t, False)
            do_superstep(t, True)
            return carry

        pltpu.sync_copy(ei_hbm.at[0, pl.ds(start, SW)], srcA)
        pltpu.sync_copy(ei_hbm.at[1, pl.ds(start, SW)], dstA)
        lax.fori_loop(0, WPT // SW // 2, pair, 0)

        do_scatter(1, dstB.at[SW - 1])
        drain_scatter(0)
        drain_scatter(1)

        plsc.subcore_barrier()

        @pl.when(sid < 15)
        def _():
            pltpu.sync_copy(acc.at[pl.ds(sid * 640, 640)],
                            out.at[pl.ds(cid * N + sid * 640, 640)])
            if with_deg:
                pltpu.sync_copy(dacc.at[pl.ds(sid * 640, 640)], zdeg)
                pltpu.sync_copy(zdeg,
                                deg_out.at[pl.ds(cid * N + sid * 640, 640)])

        @pl.when(sid == 15)
        def _():
            pltpu.sync_copy(acc.at[pl.ds(9600, 400)],
                            out.at[pl.ds(cid * N + 9600, 400)])
            if with_deg:
                pltpu.sync_copy(dacc.at[pl.ds(9600, 400)],
                                zdeg.at[pl.ds(0, 400)])
                pltpu.sync_copy(zdeg.at[pl.ds(0, 400)],
                                deg_out.at[pl.ds(cid * N + 9600, 400)])

    return pl.kernel(body, mesh=mesh, out_type=out_type, scratch_types=scratch)


_segsum_deg = _make_segsum(True)
_segsum = _make_segsum(False)


_CD = (((1,), (1,)), ((), ()))


def _tc1_body(a0, a1, d0, d1, x, wl, wr, b, o):
    deg = jnp.maximum(d0[...] + d1[...], 1.0)
    mean = (a0[...] + a1[...]) / deg
    y = lax.dot_general(mean, wl[...], _CD, preferred_element_type=jnp.float32)
    y = y + lax.dot_general(x[...], wr[...], _CD, preferred_element_type=jnp.float32)
    o[...] = jnp.maximum(y + b[...], 0.0)


def _tc2_body(m0, m1, d0, d1, x1, wl, wr, b, wa, wb, bl, o):
    deg = jnp.maximum(d0[...] + d1[...], 1.0)
    mean = (m0[...] + m1[...]) / deg
    y = lax.dot_general(mean, wl[...], _CD, preferred_element_type=jnp.float32)
    y = y + lax.dot_general(x1[...], wr[...], _CD, preferred_element_type=jnp.float32)
    x2 = jnp.maximum(y + b[...], 0.0)
    z = lax.dot_general(x1[...], wa[...], _CD, preferred_element_type=jnp.float32)
    z = z + lax.dot_general(x2, wb[...], _CD, preferred_element_type=jnp.float32)
    o[...] = z + bl[...]


BN = 2000
NB = N // BN


def _row_spec(off=0):
    return pl.BlockSpec((BN, D), lambda i, o=off: (i + o, 0))


def _deg_spec(off=0):
    return pl.BlockSpec((BN, 1), lambda i, o=off: (i + o, 0))


def _w_spec():
    return pl.BlockSpec((D, D), lambda i: (0, 0))


def _b_spec():
    return pl.BlockSpec((1, D), lambda i: (0, 0))


def _tc1(sums, deg2, x, wl, wr, b):
    return pl.pallas_call(
        _tc1_body,
        grid=(NB,),
        in_specs=[_row_spec(), _row_spec(NB), _deg_spec(), _deg_spec(NB),
                  _row_spec(), _w_spec(), _w_spec(), _b_spec()],
        out_specs=pl.BlockSpec((BN, D), lambda i: (i, 0)),
        out_shape=jax.ShapeDtypeStruct((N, D), jnp.float32),
    )(sums, sums, deg2, deg2, x, wl, wr, b)


def _tc2(sums2, deg2, x1, wl, wr, b, wa, wb, bl):
    return pl.pallas_call(
        _tc2_body,
        grid=(NB,),
        in_specs=[_row_spec(), _row_spec(NB), _deg_spec(), _deg_spec(NB),
                  _row_spec(), _w_spec(), _w_spec(), _b_spec(),
                  _w_spec(), _w_spec(), _b_spec()],
        out_specs=pl.BlockSpec((BN, D), lambda i: (i, 0)),
        out_shape=jax.ShapeDtypeStruct((N, D), jnp.float32),
    )(sums2, sums2, deg2, deg2, x1, wl, wr, b, wa, wb, bl)


def kernel(x, edge_index, W1_l, b1_l, W1_r, W2_l, b2_l, W2_r, W_lin, b_lin):
    pidx = jnp.arange(EP - E, dtype=jnp.int32)
    pad = jnp.stack([pidx % N, N + (pidx % 8)])
    ei = jnp.concatenate([edge_index.astype(jnp.int32), pad],
                         axis=1).reshape(2, WROWS, K)

    sums, deg = _segsum_deg(x, ei)
    deg2 = deg[:, None]

    x1 = _tc1(sums, deg2, x, W1_l, W1_r, b1_l[None, :])

    sums2, = _segsum(x1, ei)

    out = _tc2(sums2, deg2, x1, W2_l, W2_r, b2_l[None, :],
               W_lin[:, :D], W_lin[:, D:], b_lin[None, :])
    return out
